# baseline (speedup 1.0000x reference)
import functools

import jax
import jax.numpy as jnp
from jax import lax
from jax.experimental import pallas as pl
from jax.experimental.pallas import tpu as pltpu
from jax.experimental.pallas import tpu_sc as plsc

E = 800000
CPAD = 16

NC = 2
NS = 16
NW = NC * NS

IDX_PER_STREAM = 128
STREAMS_PER_GROUP = 16
CHUNK = IDX_PER_STREAM * STREAMS_PER_GROUP

EB = 6400
RB = EB // 8
GRID = E // EB

ESLOT = 819200
N_IDX_PAD = 4 * ESLOT
NG = N_IDX_PAD // (NW * CHUNK)
N_GROUPS = NW * NG
SLOT_BLOCKS = ESLOT // EB


NPAIR = NG // 2


def _sc_gather(table, idx):
    mesh = plsc.VectorSubcoreMesh(core_axis_name="c", subcore_axis_name="s")

    @functools.partial(
        pl.kernel,
        mesh=mesh,
        out_type=jax.ShapeDtypeStruct((N_GROUPS, STREAMS_PER_GROUP,
                                       IDX_PER_STREAM, CPAD), jnp.float32),
        scratch_types=[
            pltpu.VMEM((2, STREAMS_PER_GROUP, IDX_PER_STREAM), jnp.int32),
            pltpu.VMEM((2, STREAMS_PER_GROUP, IDX_PER_STREAM, CPAD),
                       jnp.float32),
            pltpu.SemaphoreType.DMA,
            pltpu.SemaphoreType.DMA,
            pltpu.SemaphoreType.DMA,
            pltpu.SemaphoreType.DMA,
            pltpu.SemaphoreType.DMA,
            pltpu.SemaphoreType.DMA,
        ],
        compiler_params=pltpu.CompilerParams(use_tc_tiling_on_sc=False),
    )
    def k(table_hbm, idx_hbm, out_hbm, idx_v, rows_v,
          isem0, isem1, gsem0, gsem1, wsem0, wsem1):
        wid = lax.axis_index("s") * NC + lax.axis_index("c")
        isem = (isem0, isem1)
        gsem = (gsem0, gsem1)
        wsem = (wsem0, wsem1)

        def fire(b):
            for j in range(STREAMS_PER_GROUP):
                pltpu.async_copy(table_hbm.at[idx_v.at[b, j]],
                                 rows_v.at[b, j], gsem[b])

        def drain_rows(sem):
            pltpu.make_async_copy(out_hbm.at[0], rows_v.at[0], sem).wait()

        def drain_idx(sem):
            pltpu.make_async_copy(idx_hbm.at[wid, 0], idx_v.at[0], sem).wait()

        def body(t, _):
            g0 = 2 * t

            @pl.when(t > 0)
            def _prev0():
                drain_rows(wsem[0])
                drain_idx(isem[0])

            @pl.when(t == 0)
            def _first0():
                pltpu.sync_copy(idx_hbm.at[wid, g0], idx_v.at[0])

            fire(0)

            @pl.when(t > 0)
            def _prev1():
                drain_rows(gsem[1])
                pltpu.async_copy(idx_hbm.at[wid, g0 + 1], idx_v.at[1],
                                 isem[1])
                pltpu.async_copy(rows_v.at[1], out_hbm.at[wid * NG + g0 - 1],
                                 wsem[1])
                drain_rows(wsem[1])
                drain_idx(isem[1])

            @pl.when(t == 0)
            def _first1():
                pltpu.sync_copy(idx_hbm.at[wid, g0 + 1], idx_v.at[1])

            fire(1)

            drain_rows(gsem[0])

            @pl.when(t < NPAIR - 1)
            def _prefetch():
                pltpu.async_copy(idx_hbm.at[wid, g0 + 2], idx_v.at[0],
                                 isem[0])

            pltpu.async_copy(rows_v.at[0], out_hbm.at[wid * NG + g0], wsem[0])
            return _

        lax.fori_loop(0, NPAIR, body, None)
        drain_rows(gsem[1])
        pltpu.async_copy(rows_v.at[1], out_hbm.at[wid * NG + NG - 1], wsem[1])
        drain_rows(wsem[1])
        drain_rows(wsem[0])

    return k(table, idx)


def _conv_body(x_ref, n1_ref, n2_ref, n3_ref, n4_ref, w_ref, b_ref, o_ref):
    x = x_ref[...]
    n1 = n1_ref[...]
    n2 = n2_ref[...]
    n3 = n3_ref[...]
    n4 = n4_ref[...]
    g = jnp.concatenate(
        [x, n1 + n3, n2 + n4, jnp.abs(n1 - n3), jnp.abs(n2 - n4)], axis=1
    )
    acc = jnp.dot(g, w_ref[...], preferred_element_type=jnp.float32)
    o_ref[...] = jnp.maximum(acc + b_ref[...], 0.0)


def _tc_conv(xp, nbrp, wbig, bvec):
    def nbr_spec(k):
        return pl.BlockSpec((RB, 128), lambda i, k=k: (k * SLOT_BLOCKS + i, 0))

    return pl.pallas_call(
        _conv_body,
        grid=(GRID,),
        in_specs=[
            pl.BlockSpec((RB, 128), lambda i: (i, 0)),
            nbr_spec(0),
            nbr_spec(1),
            nbr_spec(2),
            nbr_spec(3),
            pl.BlockSpec((5 * 128, 128), lambda i: (0, 0)),
            pl.BlockSpec((1, 128), lambda i: (0, 0)),
        ],
        out_specs=pl.BlockSpec((RB, 128), lambda i: (i, 0)),
        out_shape=jax.ShapeDtypeStruct((E // 8, 128), jnp.float32),
    )(xp, nbrp, nbrp, nbrp, nbrp, wbig, bvec)


def _prep_w(W, C):
    wk = jnp.transpose(W[:, :, 0, :], (2, 1, 0))
    wk = jnp.pad(wk, ((0, 0), (0, CPAD - C), (0, 0)))
    eye8 = jnp.eye(8, dtype=W.dtype)
    wblk = jax.vmap(lambda a: jnp.kron(eye8, a))(wk)
    return wblk.reshape(5 * 128, 128)


def kernel(fe, gemm_edges, W1, b1, W2, b2):
    c_in = fe.shape[1]
    fe16 = jnp.pad(fe[0], ((0, CPAD - c_in), (0, 0)))
    xp = jnp.transpose(fe16.reshape(CPAD, E // 8, 8),
                       (1, 2, 0)).reshape(E // 8, 128)

    ge_t = jnp.transpose(gemm_edges[0])
    slot_pad = (jnp.arange(ESLOT - E, dtype=jnp.int32) * 997) % E
    ge_t = jnp.concatenate(
        [ge_t, jnp.broadcast_to(slot_pad, (4, ESLOT - E))], axis=1)
    idx = ge_t.reshape(NW, NG, STREAMS_PER_GROUP, IDX_PER_STREAM)

    wb1 = _prep_w(W1, c_in)
    wb2 = _prep_w(W2, CPAD)
    b1v = jnp.tile(b1.reshape(1, CPAD), (1, 8))
    b2v = jnp.tile(b2.reshape(1, CPAD), (1, 8))

    nbr1 = _sc_gather(xp.reshape(E, CPAD), idx).reshape(N_IDX_PAD // 8, 128)
    y1p = _tc_conv(xp, nbr1, wb1, b1v)
    nbr2 = _sc_gather(y1p.reshape(E, CPAD), idx).reshape(N_IDX_PAD // 8, 128)
    y2p = _tc_conv(y1p, nbr2, wb2, b2v)
    return jnp.transpose(y2p.reshape(E // 8, 8, CPAD),
                         (2, 0, 1)).reshape(CPAD, E)[None]


# speedup vs baseline: 1.4817x; 1.4817x over previous
import functools

import jax
import jax.numpy as jnp
from jax import lax
from jax.experimental import pallas as pl
from jax.experimental.pallas import tpu as pltpu
from jax.experimental.pallas import tpu_sc as plsc

E = 800000
CPAD = 16

NC = 2
NS = 16
NW = NC * NS

IDX_PER_STREAM = 128
STREAMS_PER_GROUP = 16
CHUNK = IDX_PER_STREAM * STREAMS_PER_GROUP

EB = 6400
RB = EB // 8
GRID = E // EB

ESLOT = 819200
N_IDX_PAD = 4 * ESLOT
NG = N_IDX_PAD // (NW * CHUNK)
N_GROUPS = NW * NG
SLOT_BLOCKS = ESLOT // EB


NPAIR = NG // 2


def _sc_gather(table, idx):
    mesh = plsc.VectorSubcoreMesh(core_axis_name="c", subcore_axis_name="s")

    @functools.partial(
        pl.kernel,
        mesh=mesh,
        out_type=jax.ShapeDtypeStruct((N_GROUPS, STREAMS_PER_GROUP,
                                       IDX_PER_STREAM, CPAD), jnp.float32),
        scratch_types=[
            pltpu.VMEM((2, STREAMS_PER_GROUP, IDX_PER_STREAM), jnp.int32),
            pltpu.VMEM((2, STREAMS_PER_GROUP, IDX_PER_STREAM, CPAD),
                       jnp.float32),
            pltpu.SemaphoreType.DMA,
            pltpu.SemaphoreType.DMA,
            pltpu.SemaphoreType.DMA,
            pltpu.SemaphoreType.DMA,
            pltpu.SemaphoreType.DMA,
            pltpu.SemaphoreType.DMA,
        ],
        compiler_params=pltpu.CompilerParams(use_tc_tiling_on_sc=False),
    )
    def k(table_hbm, idx_hbm, out_hbm, idx_v, rows_v,
          isem0, isem1, gsem0, gsem1, wsem0, wsem1):
        wid = lax.axis_index("s") * NC + lax.axis_index("c")
        isem = (isem0, isem1)
        gsem = (gsem0, gsem1)
        wsem = (wsem0, wsem1)

        def fire(b):
            for j in range(STREAMS_PER_GROUP):
                pltpu.async_copy(table_hbm.at[idx_v.at[b, j]],
                                 rows_v.at[b, j], gsem[b])

        def drain_rows(sem):
            pltpu.make_async_copy(out_hbm.at[0], rows_v.at[0], sem).wait()

        def drain_idx(sem):
            pltpu.make_async_copy(idx_hbm.at[wid, 0], idx_v.at[0], sem).wait()

        def body(t, _):
            g0 = 2 * t

            @pl.when(t > 0)
            def _prev0():
                drain_rows(wsem[0])
                drain_idx(isem[0])

            @pl.when(t == 0)
            def _first0():
                pltpu.sync_copy(idx_hbm.at[wid, g0], idx_v.at[0])

            fire(0)

            @pl.when(t > 0)
            def _prev1():
                drain_rows(gsem[1])
                pltpu.async_copy(idx_hbm.at[wid, g0 + 1], idx_v.at[1],
                                 isem[1])
                pltpu.async_copy(rows_v.at[1], out_hbm.at[wid * NG + g0 - 1],
                                 wsem[1])
                drain_rows(wsem[1])
                drain_idx(isem[1])

            @pl.when(t == 0)
            def _first1():
                pltpu.sync_copy(idx_hbm.at[wid, g0 + 1], idx_v.at[1])

            fire(1)

            drain_rows(gsem[0])

            @pl.when(t < NPAIR - 1)
            def _prefetch():
                pltpu.async_copy(idx_hbm.at[wid, g0 + 2], idx_v.at[0],
                                 isem[0])

            pltpu.async_copy(rows_v.at[0], out_hbm.at[wid * NG + g0], wsem[0])
            return _

        lax.fori_loop(0, NPAIR, body, None)
        drain_rows(gsem[1])
        pltpu.async_copy(rows_v.at[1], out_hbm.at[wid * NG + NG - 1], wsem[1])
        drain_rows(wsem[1])
        drain_rows(wsem[0])

    return k(table, idx)


def _pack_body(x_ref, o_ref):
    x = x_ref[...]
    xpad = jnp.pad(x, ((0, CPAD - x.shape[0]), (0, 0)))
    t = jnp.transpose(xpad).reshape(RB, 8, CPAD)
    o_ref[...] = jnp.concatenate([t[:, j, :] for j in range(8)], axis=-1)


def _tc_pack(fe2d):
    c_in = fe2d.shape[0]
    return pl.pallas_call(
        _pack_body,
        grid=(GRID,),
        in_specs=[pl.BlockSpec((c_in, EB), lambda i: (0, i))],
        out_specs=pl.BlockSpec((RB, 128), lambda i: (i, 0)),
        out_shape=jax.ShapeDtypeStruct((E // 8, 128), jnp.float32),
    )(fe2d)


def _conv_body(x_ref, n1_ref, n2_ref, n3_ref, n4_ref, w_ref, b_ref, o_ref):
    x = x_ref[...]
    n1 = n1_ref[...]
    n2 = n2_ref[...]
    n3 = n3_ref[...]
    n4 = n4_ref[...]
    g = jnp.concatenate(
        [x, n1 + n3, n2 + n4, jnp.abs(n1 - n3), jnp.abs(n2 - n4)], axis=1
    )
    acc = jnp.dot(g, w_ref[...], preferred_element_type=jnp.float32)
    o_ref[...] = jnp.maximum(acc + b_ref[...], 0.0)


def _tc_conv(xp, nbrp, wbig, bvec):
    def nbr_spec(k):
        return pl.BlockSpec((RB, 128), lambda i, k=k: (k * SLOT_BLOCKS + i, 0))

    return pl.pallas_call(
        _conv_body,
        grid=(GRID,),
        in_specs=[
            pl.BlockSpec((RB, 128), lambda i: (i, 0)),
            nbr_spec(0),
            nbr_spec(1),
            nbr_spec(2),
            nbr_spec(3),
            pl.BlockSpec((5 * 128, 128), lambda i: (0, 0)),
            pl.BlockSpec((1, 128), lambda i: (0, 0)),
        ],
        out_specs=pl.BlockSpec((RB, 128), lambda i: (i, 0)),
        out_shape=jax.ShapeDtypeStruct((E // 8, 128), jnp.float32),
    )(xp, nbrp, nbrp, nbrp, nbrp, wbig, bvec)


def _prep_w(W, C):
    wk = jnp.transpose(W[:, :, 0, :], (2, 1, 0))
    wk = jnp.pad(wk, ((0, 0), (0, CPAD - C), (0, 0)))
    eye8 = jnp.eye(8, dtype=W.dtype)
    wblk = jax.vmap(lambda a: jnp.kron(eye8, a))(wk)
    return wblk.reshape(5 * 128, 128)


def kernel(fe, gemm_edges, W1, b1, W2, b2):
    c_in = fe.shape[1]
    xp = _tc_pack(fe[0])

    ge_t = jnp.transpose(gemm_edges[0])
    slot_pad = (jnp.arange(ESLOT - E, dtype=jnp.int32) * 997) % E
    ge_t = jnp.concatenate(
        [ge_t, jnp.broadcast_to(slot_pad, (4, ESLOT - E))], axis=1)
    idx = ge_t.reshape(NW, NG, STREAMS_PER_GROUP, IDX_PER_STREAM)

    wb1 = _prep_w(W1, c_in)
    wb2 = _prep_w(W2, CPAD)
    b1v = jnp.tile(b1.reshape(1, CPAD), (1, 8))
    b2v = jnp.tile(b2.reshape(1, CPAD), (1, 8))

    nbr1 = _sc_gather(xp.reshape(E, CPAD), idx).reshape(N_IDX_PAD // 8, 128)
    y1p = _tc_conv(xp, nbr1, wb1, b1v)
    nbr2 = _sc_gather(y1p.reshape(E, CPAD), idx).reshape(N_IDX_PAD // 8, 128)
    y2p = _tc_conv(y1p, nbr2, wb2, b2v)
    return jnp.transpose(y2p.reshape(E // 8, 8, CPAD),
                         (2, 0, 1)).reshape(CPAD, E)[None]
